# baseline (speedup 1.0000x reference)
import functools

import jax
import jax.numpy as jnp
from jax import lax
from jax.experimental import pallas as pl
from jax.experimental.pallas import tpu as pltpu
from jax.experimental.pallas import tpu_sc as plsc

NC = 2
NS = 16
NW = NC * NS
V = 1000000
D = 64
B4 = 4096
K26 = 26
B_ROWS = B4 * K26
EW = B_ROWS // NW
NG1 = EW // 16
RPT = V // NW
BLK = 512
NBLK = 64
NKEY = 64
TRASH = NKEY - 1
CAPC = 48
ECAP = CAPC * 128
DUMP = 26
OROWS = B4 * 32

_mesh = plsc.VectorSubcoreMesh(core_axis_name="c", subcore_axis_name="s")
_params = pltpu.CompilerParams(needs_layout_passes=False)


def _i16():
    return lax.iota(jnp.int32, 16)


def _splat(x):
    return jnp.full((16,), x, jnp.int32)


@functools.partial(
    pl.kernel,
    mesh=_mesh,
    out_type=(
        jax.ShapeDtypeStruct((NW, EW), jnp.int32),
        jax.ShapeDtypeStruct((NW, EW), jnp.int32),
        jax.ShapeDtypeStruct((NW, 128), jnp.int32),
    ),
    scratch_types=[
        pltpu.VMEM((EW,), jnp.int32),
        pltpu.VMEM((EW,), jnp.int32),
        pltpu.VMEM((EW,), jnp.int32),
        pltpu.VMEM((EW,), jnp.int32),
        pltpu.VMEM((512,), jnp.int32),
        pltpu.VMEM((512,), jnp.int32),
        pltpu.VMEM((512,), jnp.int32),
        pltpu.VMEM((48,), jnp.int32),
        pltpu.SemaphoreType.DMA,
        pltpu.SemaphoreType.DMA,
    ],
    compiler_params=_params,
)
def _bucketize(idx_hbm, pmap_hbm, bi_hbm, bp_hbm, pre_hbm,
               iv, pv, sbi, sbp, hist, lanepre, cnt2, prebuf, sem, sem2):
    w = lax.axis_index("s") * NC + lax.axis_index("c")
    cp1 = pltpu.async_copy(idx_hbm.at[w], iv, sem)
    cp2 = pltpu.async_copy(pmap_hbm.at[w], pv, sem2)
    cp1.wait()
    cp2.wait()

    z = _i16() * 0
    for i in range(32):
        hist[pl.ds(16 * i, 16)] = z
        cnt2[pl.ds(16 * i, 16)] = z

    def pass_a(g, c):
        ivg = iv[pl.ds(16 * g, 16)]
        flat = _i16() * 32 + ivg // RPT
        h = plsc.load_gather(hist, [flat])
        plsc.store_scatter(hist, [flat], h + 1)
        return c

    lax.fori_loop(0, NG1, pass_a, jnp.int32(0))

    run0 = z
    run1 = z
    for l in range(16):
        lanepre[pl.ds(32 * l, 16)] = run0
        lanepre[pl.ds(32 * l + 16, 16)] = run1
        run0 = run0 + hist[pl.ds(32 * l, 16)]
        run1 = run1 + hist[pl.ds(32 * l + 16, 16)]
    cs0 = jnp.cumsum(run0)
    s0 = cs0[15]
    cs1 = jnp.cumsum(run1)
    prebuf[pl.ds(0, 16)] = cs0 - run0
    prebuf[pl.ds(16, 16)] = cs1 - run1 + s0
    prebuf[pl.ds(32, 16)] = _splat(EW)

    def pass_b(g, c):
        ivg = iv[pl.ds(16 * g, 16)]
        pvg = pv[pl.ds(16 * g, 16)]
        o = ivg // RPT
        flat = _i16() * 32 + o
        lp = plsc.load_gather(lanepre, [flat])
        gp = plsc.load_gather(prebuf, [o])
        c2 = plsc.load_gather(cnt2, [flat])
        dst = gp + lp + c2
        plsc.store_scatter(sbi, [dst], ivg)
        plsc.store_scatter(sbp, [dst], pvg)
        plsc.store_scatter(cnt2, [flat], c2 + 1)
        return c

    lax.fori_loop(0, NG1, pass_b, jnp.int32(0))

    cp3 = pltpu.async_copy(sbi, bi_hbm.at[w], sem)
    cp4 = pltpu.async_copy(sbp, bp_hbm.at[w], sem2)
    cp3.wait()
    cp4.wait()
    pltpu.sync_copy(prebuf, pre_hbm.at[w, pl.ds(0, 48)])


@functools.partial(
    pl.kernel,
    mesh=_mesh,
    out_type=jax.ShapeDtypeStruct((OROWS, 128), jnp.float32),
    scratch_types=[
        pltpu.VMEM((128, BLK), jnp.float32),
        pltpu.VMEM((ECAP + 16,), jnp.int32),
        pltpu.VMEM((ECAP + 16,), jnp.int32),
        pltpu.VMEM((ECAP + 16,), jnp.int32),
        pltpu.VMEM((ECAP + 16,), jnp.int32),
        pltpu.VMEM((128, 128), jnp.float32),
        pltpu.VMEM((2, 64), jnp.int32),
        pltpu.VMEM((32, 128), jnp.int32),
        pltpu.VMEM((1024,), jnp.int32),
        pltpu.VMEM((1024,), jnp.int32),
        pltpu.VMEM((1024,), jnp.int32),
        pltpu.VMEM((96,), jnp.int32),
        pltpu.VMEM((48,), jnp.int32),
        pltpu.VMEM((48,), jnp.int32),
        pltpu.VMEM((128,), jnp.int32),
        pltpu.VMEM((64, 64), jnp.float32),
        pltpu.SMEM((8,), jnp.int32),
        pltpu.SemaphoreType.DMA,
        pltpu.SemaphoreType.DMA,
        pltpu.SemaphoreType.DMA,
        pltpu.SemaphoreType.DMA,
        pltpu.SemaphoreType.DMA,
    ],
    compiler_params=_params,
)
def _gather_stream(tT_hbm, tail_hbm, bi_hbm, bp_hbm, pre_hbm, out_hbm,
                   blk2, ebi, ebp, sbi, sbp, stage, posbuf, prew,
                   hist, lanepre, cnt2, kpre, k0buf, nchbuf, drain, tailv,
                   scal, lsem, bsem0, bsem1, ssem, psem):
    t = lax.axis_index("s") * NC + lax.axis_index("c")
    lo = t * RPT
    hi = lo + RPT
    kb0 = lo // BLK

    cpt = pltpu.async_copy(tail_hbm, tailv, lsem)
    pltpu.async_copy(pre_hbm, prew, psem).wait()
    cpt.wait()

    for h in range(2):
        wv = _i16() + 16 * h
        a = plsc.load_gather(prew, [wv, _splat(t)])
        b = plsc.load_gather(prew, [wv, _splat(t + 1)])
        k0 = a // 128
        k1 = (b + 127) // 128
        k0buf[pl.ds(16 * h, 16)] = k0
        nchbuf[pl.ds(16 * h, 16)] = k1 - k0

    z = _i16() * 0
    for hh in range(2):
        for i in range(4):
            posbuf[hh, pl.ds(16 * i, 16)] = _splat(DUMP)
    scal[0] = 0
    scal[1] = 0
    scal[2] = 0

    def get1(ref, pos):
        return ref[pl.ds(pos, 16)][0]

    def batch_cond(wc):
        return wc < NW

    def batch_body(wc):
        def load_cond(c3):
            wl, cur, fired = c3
            nch = lax.select(wl < NW, get1(nchbuf, lax.min(wl, NW - 1)),
                             jnp.int32(CAPC + 1))
            return jnp.logical_and(wl < NW, cur + nch <= CAPC)

        def load_body(c3):
            wl, cur, fired = c3
            k0w = get1(k0buf, wl)
            nch = get1(nchbuf, wl)

            def chunk(k, f):
                pltpu.async_copy(
                    bi_hbm.at[wl, pl.ds((k0w + k) * 128, 128)],
                    ebi.at[pl.ds((cur + k) * 128, 128)], lsem)
                pltpu.async_copy(
                    bp_hbm.at[wl, pl.ds((k0w + k) * 128, 128)],
                    ebp.at[pl.ds((cur + k) * 128, 128)], lsem)
                return f + 2

            fired = lax.fori_loop(0, nch, chunk, fired)
            return (wl + 1, cur + nch, fired)

        wend, curc, fired = lax.while_loop(
            load_cond, load_body, (wc, jnp.int32(0), jnp.int32(0)))

        def drain_one(k, c):
            pltpu.make_async_copy(bi_hbm.at[0, pl.ds(0, 128)], drain,
                                  lsem).wait()
            return c

        lax.fori_loop(0, fired, drain_one, jnp.int32(0))
        nent = curc * 128

        for i in range(64):
            hist[pl.ds(16 * i, 16)] = z
            cnt2[pl.ds(16 * i, 16)] = z

        def key_of(ivg):
            valid = jnp.logical_and(ivg >= lo, ivg < hi)
            return jnp.where(valid, ivg // BLK - kb0, TRASH)

        def s_pass_a(g, c):
            ivg = ebi[pl.ds(16 * g, 16)]
            flat = _i16() * NKEY + key_of(ivg)
            h = plsc.load_gather(hist, [flat])
            plsc.store_scatter(hist, [flat], h + 1)
            return c

        lax.fori_loop(0, nent // 16, s_pass_a, jnp.int32(0))

        runs = [z] * 4
        for l in range(16):
            for q in range(4):
                lanepre[pl.ds(NKEY * l + 16 * q, 16)] = runs[q]
                runs[q] = runs[q] + hist[pl.ds(NKEY * l + 16 * q, 16)]
        carry = jnp.int32(0)
        for q in range(4):
            cs = jnp.cumsum(runs[q])
            kpre[pl.ds(16 * q, 16)] = cs - runs[q] + carry
            carry = carry + cs[15]
        kpre[pl.ds(64, 16)] = _splat(0) + carry

        def s_pass_b(g, c):
            ivg = ebi[pl.ds(16 * g, 16)]
            pvg = ebp[pl.ds(16 * g, 16)]
            key = key_of(ivg)
            flat = _i16() * NKEY + key
            lp = plsc.load_gather(lanepre, [flat])
            gp = plsc.load_gather(kpre, [key])
            c2 = plsc.load_gather(cnt2, [flat])
            dst = gp + lp + c2
            plsc.store_scatter(sbi, [dst], ivg)
            plsc.store_scatter(sbp, [dst], pvg)
            plsc.store_scatter(cnt2, [flat], c2 + 1)
            return c

        lax.fori_loop(0, nent // 16, s_pass_b, jnp.int32(0))

        def startc_of(b):
            return pl.multiple_of((kb0 + b) * BLK, 128)

        def valid_blk(b):
            return (kb0 + b) * BLK < hi

        def is_tail(b):
            return (kb0 + b) * BLK + BLK > V

        def fire(b, slot):
            sem = bsem0 if slot == 0 else bsem1
            for tr in range(8):
                pltpu.async_copy(
                    tT_hbm.at[pl.ds(8 * tr, 8), pl.ds(startc_of(b), BLK)],
                    blk2.at[pl.ds(slot * 64 + 8 * tr, 8)], sem)

        def wait_blk(b, slot):
            sem = bsem0 if slot == 0 else bsem1
            for tr in range(8):
                pltpu.make_async_copy(
                    tT_hbm.at[pl.ds(8 * tr, 8), pl.ds(startc_of(b), BLK)],
                    blk2.at[pl.ds(slot * 64 + 8 * tr, 8)], sem).wait()

        def process(b, slot, tail):
            e0 = get1(kpre, b)
            e1 = get1(kpre, b + 1)
            startc = startc_of(b)

            def grp(g, c):
                off = e0 + 16 * g
                ivg = sbi[pl.ds(off, 16)]
                pvg = sbp[pl.ds(off, 16)]
                rem = e1 - off
                msk = _i16() < rem
                col = jnp.clip(ivg - startc, 0, (64 if tail else BLK) - 1)
                pos = jnp.where(msk, pvg, DUMP)

                half = scal[1]
                fill = scal[0]
                srow = half * 64 + fill * 16
                posbuf[half, pl.ds(fill * 16, 16)] = pos
                for e in range(16):
                    cev = _splat(0) + col[e]
                    rbase = 0 if tail else slot * 64
                    for gg in range(4):
                        v = plsc.load_gather(
                            tailv if tail else blk2,
                            [_i16() + (rbase + 16 * gg), cev])
                        stage[srow + e, pl.ds(16 * gg, 16)] = v
                scal[0] = fill + 1

                @pl.when(scal[0] == 4)
                def _():
                    h2 = scal[1]
                    pltpu.async_copy(
                        stage.at[pl.ds(h2 * 64, 64)],
                        out_hbm.at[posbuf.at[h2]],
                        ssem)
                    scal[2] = scal[2] + 1
                    scal[1] = 1 - h2
                    scal[0] = 0

                    @pl.when(scal[2] >= 2)
                    def _():
                        pltpu.make_async_copy(
                            out_hbm.at[pl.ds(0, 64)],
                            stage.at[pl.ds(0, 64)], ssem).wait()
                        scal[2] = scal[2] - 1
                return c

            ngr = (e1 - e0 + 15) // 16
            lax.fori_loop(0, ngr, grp, jnp.int32(0))

        def full_blk(b):
            return jnp.logical_and(valid_blk(b), jnp.logical_not(is_tail(b)))

        def tail_blk(b):
            return jnp.logical_and(valid_blk(b), is_tail(b))

        @pl.when(full_blk(0))
        def _():
            fire(0, 0)

        def pair(p, c):
            b0 = 2 * p

            @pl.when(full_blk(b0 + 1))
            def _():
                fire(b0 + 1, 1)

            @pl.when(full_blk(b0))
            def _():
                wait_blk(b0, 0)
                process(b0, 0, False)

            @pl.when(tail_blk(b0))
            def _():
                process(b0, 0, True)

            @pl.when(full_blk(b0 + 2))
            def _():
                fire(b0 + 2, 0)

            @pl.when(full_blk(b0 + 1))
            def _():
                wait_blk(b0 + 1, 1)
                process(b0 + 1, 1, False)

            @pl.when(tail_blk(b0 + 1))
            def _():
                process(b0 + 1, 1, True)
            return c

        lax.fori_loop(0, NBLK // 2, pair, jnp.int32(0))
        return wend

    lax.while_loop(batch_cond, batch_body, jnp.int32(0))

    h2 = scal[1]
    pltpu.async_copy(
        stage.at[pl.ds(h2 * 64, 64)],
        out_hbm.at[posbuf.at[h2]],
        ssem)
    scal[2] = scal[2] + 1

    def drain_s(k, c):
        pltpu.make_async_copy(
            out_hbm.at[pl.ds(0, 64)], stage.at[pl.ds(0, 64)], ssem).wait()
        return c

    lax.fori_loop(0, scal[2], drain_s, jnp.int32(0))


def kernel(input_tensor, indices):
    tT = input_tensor.T
    tail = tT[:, (V // 128) // 2 * 256:]
    idx = indices.astype(jnp.int32).reshape(NW, EW)
    p = jnp.arange(B_ROWS, dtype=jnp.int32)
    pmap = (32 * (p // K26) + p % K26).reshape(NW, EW)
    bi, bp, pre = _bucketize(idx, pmap)
    out128 = _gather_stream(tT, tail, bi, bp, pre)
    return out128.reshape(B4, 32, 128)[:, :K26, :D]


# speedup vs baseline: 1.0060x; 1.0060x over previous
import functools

import jax
import jax.numpy as jnp
from jax import lax
from jax.experimental import pallas as pl
from jax.experimental.pallas import tpu as pltpu
from jax.experimental.pallas import tpu_sc as plsc

NC = 2
NS = 16
NW = NC * NS
V = 1000000
D = 64
B4 = 4096
K26 = 26
B_ROWS = B4 * K26
EW = B_ROWS // NW
NG1 = EW // 16
RPT = V // NW
BLK = 512
NBLK = 64
NKEY = 64
TRASH = NKEY - 1
CAPC = 48
ECAP = CAPC * 128
DUMP = 26
OROWS = B4 * 32

_mesh = plsc.VectorSubcoreMesh(core_axis_name="c", subcore_axis_name="s")
_params = pltpu.CompilerParams(needs_layout_passes=False)


def _i16():
    return lax.iota(jnp.int32, 16)


def _splat(x):
    return jnp.full((16,), x, jnp.int32)


@functools.partial(
    pl.kernel,
    mesh=_mesh,
    out_type=(
        jax.ShapeDtypeStruct((NW, EW), jnp.int32),
        jax.ShapeDtypeStruct((NW, EW), jnp.int32),
        jax.ShapeDtypeStruct((NW, 128), jnp.int32),
    ),
    scratch_types=[
        pltpu.VMEM((EW,), jnp.int32),
        pltpu.VMEM((EW,), jnp.int32),
        pltpu.VMEM((EW,), jnp.int32),
        pltpu.VMEM((EW,), jnp.int32),
        pltpu.VMEM((512,), jnp.int32),
        pltpu.VMEM((512,), jnp.int32),
        pltpu.VMEM((512,), jnp.int32),
        pltpu.VMEM((48,), jnp.int32),
        pltpu.SemaphoreType.DMA,
        pltpu.SemaphoreType.DMA,
    ],
    compiler_params=_params,
)
def _bucketize(idx_hbm, pmap_hbm, bi_hbm, bp_hbm, pre_hbm,
               iv, pv, sbi, sbp, hist, lanepre, cnt2, prebuf, sem, sem2):
    w = lax.axis_index("s") * NC + lax.axis_index("c")
    cp1 = pltpu.async_copy(idx_hbm.at[w], iv, sem)
    cp2 = pltpu.async_copy(pmap_hbm.at[w], pv, sem2)
    cp1.wait()
    cp2.wait()

    z = _i16() * 0
    for i in range(32):
        hist[pl.ds(16 * i, 16)] = z
        cnt2[pl.ds(16 * i, 16)] = z

    def pass_a(g, c):
        ivg = iv[pl.ds(16 * g, 16)]
        flat = _i16() * 32 + ivg // RPT
        h = plsc.load_gather(hist, [flat])
        plsc.store_scatter(hist, [flat], h + 1)
        return c

    lax.fori_loop(0, NG1, pass_a, jnp.int32(0))

    run0 = z
    run1 = z
    for l in range(16):
        lanepre[pl.ds(32 * l, 16)] = run0
        lanepre[pl.ds(32 * l + 16, 16)] = run1
        run0 = run0 + hist[pl.ds(32 * l, 16)]
        run1 = run1 + hist[pl.ds(32 * l + 16, 16)]
    cs0 = jnp.cumsum(run0)
    s0 = cs0[15]
    cs1 = jnp.cumsum(run1)
    prebuf[pl.ds(0, 16)] = cs0 - run0
    prebuf[pl.ds(16, 16)] = cs1 - run1 + s0
    prebuf[pl.ds(32, 16)] = _splat(EW)

    def pass_b(g, c):
        ivg = iv[pl.ds(16 * g, 16)]
        pvg = pv[pl.ds(16 * g, 16)]
        o = ivg // RPT
        flat = _i16() * 32 + o
        lp = plsc.load_gather(lanepre, [flat])
        gp = plsc.load_gather(prebuf, [o])
        c2 = plsc.load_gather(cnt2, [flat])
        dst = gp + lp + c2
        plsc.store_scatter(sbi, [dst], ivg)
        plsc.store_scatter(sbp, [dst], pvg)
        plsc.store_scatter(cnt2, [flat], c2 + 1)
        return c

    lax.fori_loop(0, NG1, pass_b, jnp.int32(0))

    cp3 = pltpu.async_copy(sbi, bi_hbm.at[w], sem)
    cp4 = pltpu.async_copy(sbp, bp_hbm.at[w], sem2)
    cp3.wait()
    cp4.wait()
    pltpu.sync_copy(prebuf, pre_hbm.at[w, pl.ds(0, 48)])


@functools.partial(
    pl.kernel,
    mesh=_mesh,
    out_type=jax.ShapeDtypeStruct((OROWS, 128), jnp.float32),
    scratch_types=[
        pltpu.VMEM((128, BLK), jnp.float32),
        pltpu.VMEM((ECAP + 16,), jnp.int32),
        pltpu.VMEM((ECAP + 16,), jnp.int32),
        pltpu.VMEM((ECAP + 16,), jnp.int32),
        pltpu.VMEM((ECAP + 16,), jnp.int32),
        pltpu.VMEM((128, 128), jnp.float32),
        pltpu.VMEM((2, 64), jnp.int32),
        pltpu.VMEM((32, 128), jnp.int32),
        pltpu.VMEM((1024,), jnp.int32),
        pltpu.VMEM((1024,), jnp.int32),
        pltpu.VMEM((1024,), jnp.int32),
        pltpu.VMEM((96,), jnp.int32),
        pltpu.VMEM((48,), jnp.int32),
        pltpu.VMEM((48,), jnp.int32),
        pltpu.VMEM((128,), jnp.int32),
        pltpu.VMEM((64, 64), jnp.float32),
        pltpu.SMEM((8,), jnp.int32),
        pltpu.SemaphoreType.DMA,
        pltpu.SemaphoreType.DMA,
        pltpu.SemaphoreType.DMA,
        pltpu.SemaphoreType.DMA,
        pltpu.SemaphoreType.DMA,
    ],
    compiler_params=_params,
)
def _gather_stream(tT_hbm, tail_hbm, bi_hbm, bp_hbm, pre_hbm, out_hbm,
                   blk2, ebi, ebp, sbi, sbp, stage, posbuf, prew,
                   hist, lanepre, cnt2, kpre, k0buf, nchbuf, drain, tailv,
                   scal, lsem, bsem0, bsem1, ssem, psem):
    t = lax.axis_index("s") * NC + lax.axis_index("c")
    lo = t * RPT
    hi = lo + RPT
    kb0 = lo // BLK

    cpt = pltpu.async_copy(tail_hbm, tailv, lsem)
    pltpu.async_copy(pre_hbm, prew, psem).wait()
    cpt.wait()

    for h in range(2):
        wv = _i16() + 16 * h
        a = plsc.load_gather(prew, [wv, _splat(t)])
        b = plsc.load_gather(prew, [wv, _splat(t + 1)])
        k0 = a // 128
        k1 = (b + 127) // 128
        k0buf[pl.ds(16 * h, 16)] = k0
        nchbuf[pl.ds(16 * h, 16)] = k1 - k0

    z = _i16() * 0
    for hh in range(2):
        for i in range(4):
            posbuf[hh, pl.ds(16 * i, 16)] = _splat(DUMP)
    scal[0] = 0
    scal[1] = 0
    scal[2] = 0

    def get1(ref, pos):
        return ref[pl.ds(pos, 16)][0]

    def batch_cond(wc):
        return wc < NW

    def batch_body(wc):
        def load_cond(c3):
            wl, cur, fired = c3
            nch = lax.select(wl < NW, get1(nchbuf, lax.min(wl, NW - 1)),
                             jnp.int32(CAPC + 1))
            return jnp.logical_and(wl < NW, cur + nch <= CAPC)

        def load_body(c3):
            wl, cur, fired = c3
            k0w = get1(k0buf, wl)
            nch = get1(nchbuf, wl)

            def chunk(k, f):
                pltpu.async_copy(
                    bi_hbm.at[wl, pl.ds((k0w + k) * 128, 128)],
                    ebi.at[pl.ds((cur + k) * 128, 128)], lsem)
                pltpu.async_copy(
                    bp_hbm.at[wl, pl.ds((k0w + k) * 128, 128)],
                    ebp.at[pl.ds((cur + k) * 128, 128)], lsem)
                return f + 2

            fired = lax.fori_loop(0, nch, chunk, fired)
            return (wl + 1, cur + nch, fired)

        wend, curc, fired = lax.while_loop(
            load_cond, load_body, (wc, jnp.int32(0), jnp.int32(0)))

        def drain_one(k, c):
            pltpu.make_async_copy(bi_hbm.at[0, pl.ds(0, 128)], drain,
                                  lsem).wait()
            return c

        lax.fori_loop(0, fired, drain_one, jnp.int32(0))
        nent = curc * 128

        for i in range(64):
            hist[pl.ds(16 * i, 16)] = z
            cnt2[pl.ds(16 * i, 16)] = z

        def key_of(ivg):
            valid = jnp.logical_and(ivg >= lo, ivg < hi)
            return jnp.where(valid, ivg // BLK - kb0, TRASH)

        def s_pass_a(g, c):
            ivg = ebi[pl.ds(16 * g, 16)]
            flat = _i16() * NKEY + key_of(ivg)
            h = plsc.load_gather(hist, [flat])
            plsc.store_scatter(hist, [flat], h + 1)
            return c

        lax.fori_loop(0, nent // 16, s_pass_a, jnp.int32(0))

        runs = [z] * 4
        for l in range(16):
            for q in range(4):
                lanepre[pl.ds(NKEY * l + 16 * q, 16)] = runs[q]
                runs[q] = runs[q] + hist[pl.ds(NKEY * l + 16 * q, 16)]
        carry = jnp.int32(0)
        for q in range(4):
            cs = jnp.cumsum(runs[q])
            kpre[pl.ds(16 * q, 16)] = cs - runs[q] + carry
            carry = carry + cs[15]
        kpre[pl.ds(64, 16)] = _splat(0) + carry

        def s_pass_b(g, c):
            ivg = ebi[pl.ds(16 * g, 16)]
            pvg = ebp[pl.ds(16 * g, 16)]
            key = key_of(ivg)
            flat = _i16() * NKEY + key
            lp = plsc.load_gather(lanepre, [flat])
            gp = plsc.load_gather(kpre, [key])
            c2 = plsc.load_gather(cnt2, [flat])
            dst = gp + lp + c2
            plsc.store_scatter(sbi, [dst], ivg)
            plsc.store_scatter(sbp, [dst], pvg)
            plsc.store_scatter(cnt2, [flat], c2 + 1)
            return c

        lax.fori_loop(0, nent // 16, s_pass_b, jnp.int32(0))

        def startc_of(b):
            return pl.multiple_of((kb0 + b) * BLK, 128)

        def valid_blk(b):
            return (kb0 + b) * BLK < hi

        def is_tail(b):
            return (kb0 + b) * BLK + BLK > V

        def fire(b, slot):
            sem = bsem0 if slot == 0 else bsem1
            for tr in range(8):
                pltpu.async_copy(
                    tT_hbm.at[pl.ds(8 * tr, 8), pl.ds(startc_of(b), BLK)],
                    blk2.at[pl.ds(slot * 64 + 8 * tr, 8)], sem)

        def wait_blk(b, slot):
            sem = bsem0 if slot == 0 else bsem1
            for tr in range(8):
                pltpu.make_async_copy(
                    tT_hbm.at[pl.ds(8 * tr, 8), pl.ds(startc_of(b), BLK)],
                    blk2.at[pl.ds(slot * 64 + 8 * tr, 8)], sem).wait()

        def process(b, slot, tail):
            e0 = get1(kpre, b)
            e1 = get1(kpre, b + 1)
            startc = startc_of(b)

            def grp(g, c):
                off = e0 + 16 * g
                ivg = sbi[pl.ds(off, 16)]
                pvg = sbp[pl.ds(off, 16)]
                rem = e1 - off
                msk = _i16() < rem
                col = jnp.clip(ivg - startc, 0, (64 if tail else BLK) - 1)
                pos = jnp.where(msk, pvg, DUMP)

                half = scal[1]
                fill = scal[0]
                srow = half * 64 + fill * 16
                posbuf[half, pl.ds(fill * 16, 16)] = pos
                rows16 = _i16() + srow
                rbase = 0 if tail else slot * 64
                for cc in range(64):
                    v = plsc.load_gather(
                        tailv if tail else blk2,
                        [_splat(rbase + cc), col])
                    plsc.store_scatter(stage, [rows16, _splat(cc)], v)
                scal[0] = fill + 1

                @pl.when(scal[0] == 4)
                def _():
                    h2 = scal[1]
                    pltpu.async_copy(
                        stage.at[pl.ds(h2 * 64, 64)],
                        out_hbm.at[posbuf.at[h2]],
                        ssem)
                    scal[2] = scal[2] + 1
                    scal[1] = 1 - h2
                    scal[0] = 0

                    @pl.when(scal[2] >= 2)
                    def _():
                        pltpu.make_async_copy(
                            out_hbm.at[pl.ds(0, 64)],
                            stage.at[pl.ds(0, 64)], ssem).wait()
                        scal[2] = scal[2] - 1
                return c

            ngr = (e1 - e0 + 15) // 16
            lax.fori_loop(0, ngr, grp, jnp.int32(0))

        def full_blk(b):
            return jnp.logical_and(valid_blk(b), jnp.logical_not(is_tail(b)))

        def tail_blk(b):
            return jnp.logical_and(valid_blk(b), is_tail(b))

        @pl.when(full_blk(0))
        def _():
            fire(0, 0)

        def pair(p, c):
            b0 = 2 * p

            @pl.when(full_blk(b0 + 1))
            def _():
                fire(b0 + 1, 1)

            @pl.when(full_blk(b0))
            def _():
                wait_blk(b0, 0)
                process(b0, 0, False)

            @pl.when(tail_blk(b0))
            def _():
                process(b0, 0, True)

            @pl.when(full_blk(b0 + 2))
            def _():
                fire(b0 + 2, 0)

            @pl.when(full_blk(b0 + 1))
            def _():
                wait_blk(b0 + 1, 1)
                process(b0 + 1, 1, False)

            @pl.when(tail_blk(b0 + 1))
            def _():
                process(b0 + 1, 1, True)
            return c

        lax.fori_loop(0, NBLK // 2, pair, jnp.int32(0))
        return wend

    lax.while_loop(batch_cond, batch_body, jnp.int32(0))

    h2 = scal[1]
    pltpu.async_copy(
        stage.at[pl.ds(h2 * 64, 64)],
        out_hbm.at[posbuf.at[h2]],
        ssem)
    scal[2] = scal[2] + 1

    def drain_s(k, c):
        pltpu.make_async_copy(
            out_hbm.at[pl.ds(0, 64)], stage.at[pl.ds(0, 64)], ssem).wait()
        return c

    lax.fori_loop(0, scal[2], drain_s, jnp.int32(0))


def kernel(input_tensor, indices):
    tT = input_tensor.T
    tail = tT[:, (V // 128) // 2 * 256:]
    idx = indices.astype(jnp.int32).reshape(NW, EW)
    p = jnp.arange(B_ROWS, dtype=jnp.int32)
    pmap = (32 * (p // K26) + p % K26).reshape(NW, EW)
    bi, bp, pre = _bucketize(idx, pmap)
    out128 = _gather_stream(tT, tail, bi, bp, pre)
    return out128.reshape(B4, 32, 128)[:, :K26, :D]


# speedup vs baseline: 1.0070x; 1.0010x over previous
import functools

import jax
import jax.numpy as jnp
from jax import lax
from jax.experimental import pallas as pl
from jax.experimental.pallas import tpu as pltpu
from jax.experimental.pallas import tpu_sc as plsc

NC = 2
NS = 16
NW = NC * NS
V = 1000000
D = 64
B4 = 4096
K26 = 26
B_ROWS = B4 * K26
EW = B_ROWS // NW
NG1 = EW // 16
RPT = V // NW
BLK = 512
NBLK = 64
NKEY = 64
TRASH = NKEY - 1
CAPC = 48
ECAP = CAPC * 128
DUMP = 26
OROWS = B4 * 32

_mesh = plsc.VectorSubcoreMesh(core_axis_name="c", subcore_axis_name="s")
_params = pltpu.CompilerParams(needs_layout_passes=False)


def _i16():
    return lax.iota(jnp.int32, 16)


def _splat(x):
    return jnp.full((16,), x, jnp.int32)


@functools.partial(
    pl.kernel,
    mesh=_mesh,
    out_type=(
        jax.ShapeDtypeStruct((NW, EW), jnp.int32),
        jax.ShapeDtypeStruct((NW, EW), jnp.int32),
        jax.ShapeDtypeStruct((NW, 128), jnp.int32),
    ),
    scratch_types=[
        pltpu.VMEM((EW,), jnp.int32),
        pltpu.VMEM((EW,), jnp.int32),
        pltpu.VMEM((EW,), jnp.int32),
        pltpu.VMEM((EW,), jnp.int32),
        pltpu.VMEM((512,), jnp.int32),
        pltpu.VMEM((512,), jnp.int32),
        pltpu.VMEM((512,), jnp.int32),
        pltpu.VMEM((48,), jnp.int32),
        pltpu.SemaphoreType.DMA,
        pltpu.SemaphoreType.DMA,
    ],
    compiler_params=_params,
)
def _bucketize(idx_hbm, pmap_hbm, bi_hbm, bp_hbm, pre_hbm,
               iv, pv, sbi, sbp, hist, lanepre, cnt2, prebuf, sem, sem2):
    w = lax.axis_index("s") * NC + lax.axis_index("c")
    cp1 = pltpu.async_copy(idx_hbm.at[w], iv, sem)
    cp2 = pltpu.async_copy(pmap_hbm.at[w], pv, sem2)
    cp1.wait()
    cp2.wait()

    z = _i16() * 0
    for i in range(32):
        hist[pl.ds(16 * i, 16)] = z
        cnt2[pl.ds(16 * i, 16)] = z

    def pass_a(g, c):
        ivg = iv[pl.ds(16 * g, 16)]
        flat = _i16() * 32 + ivg // RPT
        h = plsc.load_gather(hist, [flat])
        plsc.store_scatter(hist, [flat], h + 1)
        return c

    lax.fori_loop(0, NG1, pass_a, jnp.int32(0))

    run0 = z
    run1 = z
    for l in range(16):
        lanepre[pl.ds(32 * l, 16)] = run0
        lanepre[pl.ds(32 * l + 16, 16)] = run1
        run0 = run0 + hist[pl.ds(32 * l, 16)]
        run1 = run1 + hist[pl.ds(32 * l + 16, 16)]
    cs0 = jnp.cumsum(run0)
    s0 = cs0[15]
    cs1 = jnp.cumsum(run1)
    prebuf[pl.ds(0, 16)] = cs0 - run0
    prebuf[pl.ds(16, 16)] = cs1 - run1 + s0
    prebuf[pl.ds(32, 16)] = _splat(EW)

    def pass_b(g, c):
        ivg = iv[pl.ds(16 * g, 16)]
        pvg = pv[pl.ds(16 * g, 16)]
        o = ivg // RPT
        flat = _i16() * 32 + o
        lp = plsc.load_gather(lanepre, [flat])
        gp = plsc.load_gather(prebuf, [o])
        c2 = plsc.load_gather(cnt2, [flat])
        dst = gp + lp + c2
        plsc.store_scatter(sbi, [dst], ivg)
        plsc.store_scatter(sbp, [dst], pvg)
        plsc.store_scatter(cnt2, [flat], c2 + 1)
        return c

    lax.fori_loop(0, NG1, pass_b, jnp.int32(0))

    cp3 = pltpu.async_copy(sbi, bi_hbm.at[w], sem)
    cp4 = pltpu.async_copy(sbp, bp_hbm.at[w], sem2)
    cp3.wait()
    cp4.wait()
    pltpu.sync_copy(prebuf, pre_hbm.at[w, pl.ds(0, 48)])


@functools.partial(
    pl.kernel,
    mesh=_mesh,
    out_type=jax.ShapeDtypeStruct((OROWS, 128), jnp.float32),
    scratch_types=[
        pltpu.VMEM((128, BLK), jnp.float32),
        pltpu.VMEM((ECAP + 16,), jnp.int32),
        pltpu.VMEM((ECAP + 16,), jnp.int32),
        pltpu.VMEM((ECAP + 16,), jnp.int32),
        pltpu.VMEM((ECAP + 16,), jnp.int32),
        pltpu.VMEM((128, 128), jnp.float32),
        pltpu.VMEM((2, 64), jnp.int32),
        pltpu.VMEM((32, 128), jnp.int32),
        pltpu.VMEM((1024,), jnp.int32),
        pltpu.VMEM((1024,), jnp.int32),
        pltpu.VMEM((1024,), jnp.int32),
        pltpu.VMEM((96,), jnp.int32),
        pltpu.VMEM((48,), jnp.int32),
        pltpu.VMEM((48,), jnp.int32),
        pltpu.VMEM((128,), jnp.int32),
        pltpu.VMEM((64, 64), jnp.float32),
        pltpu.SMEM((8,), jnp.int32),
        pltpu.SemaphoreType.DMA,
        pltpu.SemaphoreType.DMA,
        pltpu.SemaphoreType.DMA,
        pltpu.SemaphoreType.DMA,
        pltpu.SemaphoreType.DMA,
    ],
    compiler_params=_params,
)
def _gather_stream(tT_hbm, tail_hbm, bi_hbm, bp_hbm, pre_hbm, out_hbm,
                   blk2, ebi, ebp, sbi, sbp, stage, posbuf, prew,
                   hist, lanepre, cnt2, kpre, k0buf, nchbuf, drain, tailv,
                   scal, lsem, bsem0, bsem1, ssem, psem):
    t = lax.axis_index("s") * NC + lax.axis_index("c")
    lo = t * RPT
    hi = lo + RPT
    kb0 = lo // BLK

    cpt = pltpu.async_copy(tail_hbm, tailv, lsem)
    pltpu.async_copy(pre_hbm, prew, psem).wait()
    cpt.wait()

    for h in range(2):
        wv = _i16() + 16 * h
        a = plsc.load_gather(prew, [wv, _splat(t)])
        b = plsc.load_gather(prew, [wv, _splat(t + 1)])
        k0 = a // 128
        k1 = (b + 127) // 128
        k0buf[pl.ds(16 * h, 16)] = k0
        nchbuf[pl.ds(16 * h, 16)] = k1 - k0

    z = _i16() * 0
    for hh in range(2):
        for i in range(4):
            posbuf[hh, pl.ds(16 * i, 16)] = _splat(DUMP)
    scal[0] = 0
    scal[1] = 0
    scal[2] = 0

    def get1(ref, pos):
        return ref[pl.ds(pos, 16)][0]

    def batch_cond(wc):
        return wc < NW

    def batch_body(wc):
        def load_cond(c3):
            wl, cur, fired = c3
            nch = lax.select(wl < NW, get1(nchbuf, lax.min(wl, NW - 1)),
                             jnp.int32(CAPC + 1))
            return jnp.logical_and(wl < NW, cur + nch <= CAPC)

        def load_body(c3):
            wl, cur, fired = c3
            k0w = get1(k0buf, wl)
            nch = get1(nchbuf, wl)

            def chunk(k, f):
                pltpu.async_copy(
                    bi_hbm.at[wl, pl.ds((k0w + k) * 128, 128)],
                    ebi.at[pl.ds((cur + k) * 128, 128)], lsem)
                pltpu.async_copy(
                    bp_hbm.at[wl, pl.ds((k0w + k) * 128, 128)],
                    ebp.at[pl.ds((cur + k) * 128, 128)], lsem)
                return f + 2

            fired = lax.fori_loop(0, nch, chunk, fired)
            return (wl + 1, cur + nch, fired)

        wend, curc, fired = lax.while_loop(
            load_cond, load_body, (wc, jnp.int32(0), jnp.int32(0)))

        def drain_one(k, c):
            pltpu.make_async_copy(bi_hbm.at[0, pl.ds(0, 128)], drain,
                                  lsem).wait()
            return c

        lax.fori_loop(0, fired, drain_one, jnp.int32(0))
        nent = curc * 128

        for i in range(64):
            hist[pl.ds(16 * i, 16)] = z
            cnt2[pl.ds(16 * i, 16)] = z

        def key_of(ivg):
            valid = jnp.logical_and(ivg >= lo, ivg < hi)
            return jnp.where(valid, ivg // BLK - kb0, TRASH)

        def s_pass_a(g, c):
            ivg = ebi[pl.ds(16 * g, 16)]
            flat = _i16() * NKEY + key_of(ivg)
            h = plsc.load_gather(hist, [flat])
            plsc.store_scatter(hist, [flat], h + 1)
            return c

        lax.fori_loop(0, nent // 16, s_pass_a, jnp.int32(0))

        runs = [z] * 4
        for l in range(16):
            for q in range(4):
                lanepre[pl.ds(NKEY * l + 16 * q, 16)] = runs[q]
                runs[q] = runs[q] + hist[pl.ds(NKEY * l + 16 * q, 16)]
        carry = jnp.int32(0)
        for q in range(4):
            cs = jnp.cumsum(runs[q])
            kpre[pl.ds(16 * q, 16)] = cs - runs[q] + carry
            carry = carry + cs[15]
        kpre[pl.ds(64, 16)] = _splat(0) + carry

        def s_pass_b(g, c):
            ivg = ebi[pl.ds(16 * g, 16)]
            pvg = ebp[pl.ds(16 * g, 16)]
            key = key_of(ivg)
            flat = _i16() * NKEY + key
            lp = plsc.load_gather(lanepre, [flat])
            gp = plsc.load_gather(kpre, [key])
            c2 = plsc.load_gather(cnt2, [flat])
            dst = gp + lp + c2
            plsc.store_scatter(sbi, [dst], ivg)
            plsc.store_scatter(sbp, [dst], pvg)
            plsc.store_scatter(cnt2, [flat], c2 + 1)
            return c

        lax.fori_loop(0, nent // 16, s_pass_b, jnp.int32(0))

        def startc_of(b):
            return pl.multiple_of((kb0 + b) * BLK, 128)

        def valid_blk(b):
            return (kb0 + b) * BLK < hi

        def is_tail(b):
            return (kb0 + b) * BLK + BLK > V

        def fire(b, slot):
            sem = bsem0 if slot == 0 else bsem1
            for tr in range(8):
                pltpu.async_copy(
                    tT_hbm.at[pl.ds(8 * tr, 8), pl.ds(startc_of(b), BLK)],
                    blk2.at[pl.ds(slot * 64 + 8 * tr, 8)], sem)

        def wait_blk(b, slot):
            sem = bsem0 if slot == 0 else bsem1
            for tr in range(8):
                pltpu.make_async_copy(
                    tT_hbm.at[pl.ds(8 * tr, 8), pl.ds(startc_of(b), BLK)],
                    blk2.at[pl.ds(slot * 64 + 8 * tr, 8)], sem).wait()

        def process(b, slot, tail):
            e0 = get1(kpre, b)
            e1 = get1(kpre, b + 1)
            startc = startc_of(b)

            def grp(g, c):
                off = e0 + 16 * g
                ivg = sbi[pl.ds(off, 16)]
                pvg = sbp[pl.ds(off, 16)]
                rem = e1 - off
                msk = _i16() < rem
                col = jnp.clip(ivg - startc, 0, (64 if tail else BLK) - 1)
                pos = jnp.where(msk, pvg, DUMP)

                half = scal[1]
                fill = scal[0]
                srow = half * 64 + fill * 16
                posbuf[half, pl.ds(fill * 16, 16)] = pos
                rows16 = _i16() + srow
                rbase = 0 if tail else slot * 64

                def cbody(cc, cacc):
                    v = plsc.load_gather(
                        tailv if tail else blk2,
                        [_splat(rbase) + cc, col])
                    plsc.store_scatter(stage, [rows16, _splat(0) + cc], v)
                    return cacc

                lax.fori_loop(0, 64, cbody, jnp.int32(0))
                scal[0] = fill + 1

                @pl.when(scal[0] == 4)
                def _():
                    h2 = scal[1]
                    pltpu.async_copy(
                        stage.at[pl.ds(h2 * 64, 64)],
                        out_hbm.at[posbuf.at[h2]],
                        ssem)
                    scal[2] = scal[2] + 1
                    scal[1] = 1 - h2
                    scal[0] = 0

                    @pl.when(scal[2] >= 2)
                    def _():
                        pltpu.make_async_copy(
                            out_hbm.at[pl.ds(0, 64)],
                            stage.at[pl.ds(0, 64)], ssem).wait()
                        scal[2] = scal[2] - 1
                return c

            ngr = (e1 - e0 + 15) // 16
            lax.fori_loop(0, ngr, grp, jnp.int32(0))

        def full_blk(b):
            return jnp.logical_and(valid_blk(b), jnp.logical_not(is_tail(b)))

        def tail_blk(b):
            return jnp.logical_and(valid_blk(b), is_tail(b))

        @pl.when(full_blk(0))
        def _():
            fire(0, 0)

        def pair(p, c):
            b0 = 2 * p

            @pl.when(full_blk(b0 + 1))
            def _():
                fire(b0 + 1, 1)

            @pl.when(full_blk(b0))
            def _():
                wait_blk(b0, 0)
                process(b0, 0, False)

            @pl.when(tail_blk(b0))
            def _():
                process(b0, 0, True)

            @pl.when(full_blk(b0 + 2))
            def _():
                fire(b0 + 2, 0)

            @pl.when(full_blk(b0 + 1))
            def _():
                wait_blk(b0 + 1, 1)
                process(b0 + 1, 1, False)

            @pl.when(tail_blk(b0 + 1))
            def _():
                process(b0 + 1, 1, True)
            return c

        lax.fori_loop(0, NBLK // 2, pair, jnp.int32(0))
        return wend

    lax.while_loop(batch_cond, batch_body, jnp.int32(0))

    h2 = scal[1]
    pltpu.async_copy(
        stage.at[pl.ds(h2 * 64, 64)],
        out_hbm.at[posbuf.at[h2]],
        ssem)
    scal[2] = scal[2] + 1

    def drain_s(k, c):
        pltpu.make_async_copy(
            out_hbm.at[pl.ds(0, 64)], stage.at[pl.ds(0, 64)], ssem).wait()
        return c

    lax.fori_loop(0, scal[2], drain_s, jnp.int32(0))


def kernel(input_tensor, indices):
    tT = input_tensor.T
    tail = tT[:, (V // 128) // 2 * 256:]
    idx = indices.astype(jnp.int32).reshape(NW, EW)
    p = jnp.arange(B_ROWS, dtype=jnp.int32)
    pmap = (32 * (p // K26) + p % K26).reshape(NW, EW)
    bi, bp, pre = _bucketize(idx, pmap)
    out128 = _gather_stream(tT, tail, bi, bp, pre)
    return out128.reshape(B4, 32, 128)[:, :K26, :D]


# speedup vs baseline: 2.8232x; 2.8036x over previous
import functools

import jax
import jax.numpy as jnp
from jax import lax
from jax.experimental import pallas as pl
from jax.experimental.pallas import tpu as pltpu
from jax.experimental.pallas import tpu_sc as plsc

NC = 2
NS = 16
NW = NC * NS

B_ROWS = 4096 * 26
D = 64
DP = 128
B_PER_W = B_ROWS // NW
CHUNK = 128
N_CHUNKS = B_PER_W // CHUNK

_mesh = plsc.VectorSubcoreMesh(core_axis_name="c", subcore_axis_name="s")


@functools.partial(
    pl.kernel,
    mesh=_mesh,
    out_type=jax.ShapeDtypeStruct((B_ROWS, DP), jnp.float32),
    scratch_types=[
        pltpu.VMEM((N_CHUNKS, CHUNK), jnp.int32),
    ]
    + [pltpu.VMEM((CHUNK, DP), jnp.float32) for _ in range(4)]
    + [pltpu.SemaphoreType.DMA for _ in range(8)],
)
def _gather_sc(table_hbm, idx_hbm, out_hbm, idx_v, *scratch):
    NBUF = 4
    W_LAG = 2
    rows = scratch[:NBUF]
    gsem = scratch[NBUF:2 * NBUF]
    wsem = scratch[2 * NBUF:3 * NBUF]

    wid = lax.axis_index("s") * NC + lax.axis_index("c")
    base = wid * B_PER_W
    pltpu.sync_copy(idx_hbm.at[wid], idx_v)

    def start_gather(j, b):
        return pltpu.async_copy(table_hbm.at[idx_v.at[j]], rows[b], gsem[b])

    def start_write(j, b):
        return pltpu.async_copy(
            rows[b],
            out_hbm.at[pl.ds(base + j * CHUNK, CHUNK)],
            wsem[b],
        )

    gathers = {}
    writes = {}
    for b in range(NBUF):
        gathers[b] = start_gather(b, b)
    for j in range(N_CHUNKS):
        b = j % NBUF
        gathers[b].wait()
        writes[b] = start_write(j, b)
        k = j - W_LAG
        nk = k + NBUF
        if k >= 0 and nk < N_CHUNKS:
            kb = k % NBUF
            writes[kb].wait()
            gathers[kb] = start_gather(nk, kb)
    for j in range(min(N_CHUNKS - W_LAG, N_CHUNKS - NBUF), N_CHUNKS):
        writes[j % NBUF].wait()


def kernel(input_tensor, indices):
    table_padded = jnp.pad(input_tensor, ((0, 0), (0, DP - D)))
    idx = indices.reshape(NW, N_CHUNKS, CHUNK).astype(jnp.int32)
    out = _gather_sc(table_padded, idx)
    return out[:, :D].reshape(indices.shape[0], indices.shape[1], D)
